# baseline (speedup 1.0000x reference)
import functools

import jax
import jax.numpy as jnp
from jax import lax
from jax.experimental import pallas as pl
from jax.experimental.pallas import tpu as pltpu
from jax.experimental.pallas import tpu_sc as plsc

NC = 2
NS = 16
L = 16
NW = NC * NS
N_TOK = 4 * 2048
D = 768
TOK_PER_W = N_TOK // NW
BUFT = 56
CHUNKS = (56, 56, 56, 56, 32)
DV = D // L

_mesh = plsc.VectorSubcoreMesh(core_axis_name="c", subcore_axis_name="s")


@functools.partial(
    pl.kernel,
    out_type=jax.ShapeDtypeStruct((N_TOK, D), jnp.float32),
    mesh=_mesh,
    scratch_types=[
        pltpu.VMEM((TOK_PER_W,), jnp.int32),
        pltpu.VMEM((2 * TOK_PER_W,), jnp.int32),
        pltpu.VMEM((TOK_PER_W,), jnp.int32),
        pltpu.VMEM((TOK_PER_W,), jnp.int32),
        pltpu.VMEM((TOK_PER_W,), jnp.int32),
        pltpu.VMEM((TOK_PER_W,), jnp.int32),
        pltpu.VMEM((BUFT, D), jnp.float32),
        pltpu.VMEM((BUFT, D), jnp.float32),
        pltpu.VMEM((BUFT, D), jnp.float32),
        pltpu.SemaphoreType.DMA,
        pltpu.SemaphoreType.DMA,
        pltpu.SemaphoreType.DMA,
        pltpu.SemaphoreType.DMA,
    ],
)
def _embed4(ids_hbm, coords_hbm, word_hbm, p0_hbm, p1_hbm, p2_hbm, p3_hbm,
            out_hbm, idsb, cb, c0b, c1b, c2b, c3b,
            acc, tmpa, tmpb,
            sem_w, sem_a, sem_b, sem_o):
    wid = lax.axis_index("s") * NC + lax.axis_index("c")
    wbase = wid * TOK_PER_W

    pltpu.sync_copy(ids_hbm.at[pl.ds(wbase, TOK_PER_W)], idsb)
    w_first = pltpu.async_copy(
        word_hbm.at[idsb.at[pl.ds(0, CHUNKS[0])]],
        acc.at[pl.ds(0, CHUNKS[0])], sem_w)
    lane = lax.iota(jnp.int32, L)
    m0, m1, m2 = lane < 4, lane < 8, lane < 12
    idx_base = (lane & 3) * 4
    cols = (c0b, c1b, c2b, c3b)
    half_toks = TOK_PER_W // 2
    for half in range(2):
        pltpu.sync_copy(
            coords_hbm.at[pl.ds(4 * (wbase + half * half_toks),
                                4 * half_toks)], cb)
        for k in range(half_toks // L):
            srcs = [cb[pl.ds(k * 4 * L + q * L, L)] for q in range(4)]
            for t in range(4):
                idx = idx_base + t
                g = [s.at[idx].get(mode="promise_in_bounds") for s in srcs]
                v = jnp.where(m0, g[0],
                              jnp.where(m1, g[1],
                                        jnp.where(m2, g[2], g[3])))
                cols[t][pl.ds(half * half_toks + k * L, L)] = v

    def add_pass(accr, tmpr, rows):
        def row(t, c):
            for j in range(DV):
                sl = pl.ds(j * L, L)
                plsc.addupdate(accr.at[t, sl], tmpr[t, sl])
            return c
        lax.fori_loop(0, rows, row, 0)

    def final_pass(accr, tmpr, rows):
        def row(t, c):
            for j in range(DV):
                sl = pl.ds(j * L, L)
                tmpr[t, sl] = accr[t, sl] + tmpr[t, sl]
            return c
        lax.fori_loop(0, rows, row, 0)

    def chunk_body(off, n, prev=None, w_pref=None):
        acc_s = acc.at[pl.ds(0, n)]
        ta_s = tmpa.at[pl.ds(0, n)]
        tb_s = tmpb.at[pl.ds(0, n)]
        out_dst = out_hbm.at[pl.ds(wbase + off, n)]
        if w_pref is None:
            w = pltpu.async_copy(word_hbm.at[idsb.at[pl.ds(off, n)]],
                                 acc_s, sem_w)
        else:
            w = w_pref
        g0 = pltpu.async_copy(p0_hbm.at[c0b.at[pl.ds(off, n)]], ta_s, sem_a)
        if prev is not None:
            po, pn = prev
            pltpu.make_async_copy(tmpb.at[pl.ds(0, pn)],
                                  out_hbm.at[pl.ds(wbase + po, pn)],
                                  sem_o).wait()
        g1 = pltpu.async_copy(p1_hbm.at[c1b.at[pl.ds(off, n)]], tb_s, sem_b)
        w.wait()
        g0.wait()
        add_pass(acc, tmpa, n)
        g2 = pltpu.async_copy(p2_hbm.at[c2b.at[pl.ds(off, n)]], ta_s, sem_a)
        g1.wait()
        add_pass(acc, tmpb, n)
        g3 = pltpu.async_copy(p3_hbm.at[c3b.at[pl.ds(off, n)]], tb_s, sem_b)
        g2.wait()
        add_pass(acc, tmpa, n)
        g3.wait()
        final_pass(acc, tmpb, n)
        pltpu.async_copy(tb_s, out_dst, sem_o)

    off = 0
    prev = None
    for k, n in enumerate(CHUNKS):
        chunk_body(off, n, prev=prev, w_pref=w_first if k == 0 else None)
        prev = (off, n)
        off += n

    po, pn = prev
    pltpu.make_async_copy(tmpb.at[pl.ds(0, pn)],
                          out_hbm.at[pl.ds(wbase + po, pn)], sem_o).wait()


def kernel(ids, coords, word, pos0, pos1, pos2, pos3):
    B, S = ids.shape
    ids_f = ids.reshape(N_TOK).astype(jnp.int32)
    coords_f = coords.reshape(4 * N_TOK).astype(jnp.int32)
    out = _embed4(ids_f, coords_f, word, pos0, pos1, pos2, pos3)
    return out.reshape(B, S, D)


# speedup vs baseline: 1.0149x; 1.0149x over previous
import functools

import jax
import jax.numpy as jnp
from jax import lax
from jax.experimental import pallas as pl
from jax.experimental.pallas import tpu as pltpu
from jax.experimental.pallas import tpu_sc as plsc

NC = 2
NS = 16
L = 16
NW = NC * NS
N_TOK = 4 * 2048
D = 768
TOK_PER_W = N_TOK // NW
BUFT = 56
CHUNKS = (56, 56, 56, 56, 32)
DV = D // L

_mesh = plsc.VectorSubcoreMesh(core_axis_name="c", subcore_axis_name="s")


@functools.partial(
    pl.kernel,
    out_type=jax.ShapeDtypeStruct((N_TOK, D), jnp.float32),
    mesh=_mesh,
    scratch_types=[
        pltpu.VMEM((TOK_PER_W,), jnp.int32),
        pltpu.VMEM((4 * TOK_PER_W,), jnp.int32),
        pltpu.VMEM((BUFT, D), jnp.float32),
        pltpu.VMEM((BUFT, D), jnp.float32),
        pltpu.VMEM((BUFT, D), jnp.float32),
        pltpu.SemaphoreType.DMA,
        pltpu.SemaphoreType.DMA,
        pltpu.SemaphoreType.DMA,
        pltpu.SemaphoreType.DMA,
    ],
)
def _embed4(ids_hbm, cstack_hbm,
            word_hbm, p0_hbm, p1_hbm, p2_hbm, p3_hbm,
            out_hbm, idsb, cb4,
            acc, tmpa, tmpb,
            sem_w, sem_a, sem_b, sem_o):
    wid = lax.axis_index("s") * NC + lax.axis_index("c")
    wbase = wid * TOK_PER_W

    pltpu.sync_copy(ids_hbm.at[pl.ds(wbase, TOK_PER_W)], idsb)
    w_first = pltpu.async_copy(
        word_hbm.at[idsb.at[pl.ds(0, CHUNKS[0])]],
        acc.at[pl.ds(0, CHUNKS[0])], sem_w)
    pltpu.async_copy(cstack_hbm.at[pl.ds(4 * wbase, 4 * TOK_PER_W)],
                     cb4, sem_o).wait()

    def add_pass(accr, tmpr, rows):
        def row(t, c):
            for j in range(DV):
                sl = pl.ds(j * L, L)
                plsc.addupdate(accr.at[t, sl], tmpr[t, sl])
            return c
        lax.fori_loop(0, rows, row, 0)

    def final_pass(accr, tmpr, rows):
        def row(t, c):
            for j in range(DV):
                sl = pl.ds(j * L, L)
                tmpr[t, sl] = accr[t, sl] + tmpr[t, sl]
            return c
        lax.fori_loop(0, rows, row, 0)

    def chunk_body(off, n, prev=None, w_pref=None):
        acc_s = acc.at[pl.ds(0, n)]
        ta_s = tmpa.at[pl.ds(0, n)]
        tb_s = tmpb.at[pl.ds(0, n)]
        out_dst = out_hbm.at[pl.ds(wbase + off, n)]
        if w_pref is None:
            w = pltpu.async_copy(word_hbm.at[idsb.at[pl.ds(off, n)]],
                                 acc_s, sem_w)
        else:
            w = w_pref
        g0 = pltpu.async_copy(p0_hbm.at[cb4.at[pl.ds(off, n)]],
                              ta_s, sem_a)
        if prev is not None:
            po, pn = prev
            pltpu.make_async_copy(tmpb.at[pl.ds(0, pn)],
                                  out_hbm.at[pl.ds(wbase + po, pn)],
                                  sem_o).wait()
        g1 = pltpu.async_copy(p1_hbm.at[cb4.at[pl.ds(TOK_PER_W + off, n)]],
                              tb_s, sem_b)
        w.wait()
        g0.wait()
        add_pass(acc, tmpa, n)
        g2 = pltpu.async_copy(
            p2_hbm.at[cb4.at[pl.ds(2 * TOK_PER_W + off, n)]], ta_s, sem_a)
        g1.wait()
        add_pass(acc, tmpb, n)
        g3 = pltpu.async_copy(
            p3_hbm.at[cb4.at[pl.ds(3 * TOK_PER_W + off, n)]], tb_s, sem_b)
        g2.wait()
        add_pass(acc, tmpa, n)
        g3.wait()
        final_pass(acc, tmpb, n)
        pltpu.async_copy(tb_s, out_dst, sem_o)

    off = 0
    prev = None
    for k, n in enumerate(CHUNKS):
        chunk_body(off, n, prev=prev, w_pref=w_first if k == 0 else None)
        prev = (off, n)
        off += n

    po, pn = prev
    pltpu.make_async_copy(tmpb.at[pl.ds(0, pn)],
                          out_hbm.at[pl.ds(wbase + po, pn)], sem_o).wait()


def kernel(ids, coords, word, pos0, pos1, pos2, pos3):
    B, S = ids.shape
    ids_f = ids.reshape(N_TOK).astype(jnp.int32)
    cstack = (coords.reshape(NW, TOK_PER_W, 4).astype(jnp.int32)
              .transpose(0, 2, 1).reshape(4 * N_TOK))
    out = _embed4(ids_f, cstack, word, pos0, pos1, pos2, pos3)
    return out.reshape(B, S, D)


# speedup vs baseline: 1.0973x; 1.0811x over previous
import functools

import jax
import jax.numpy as jnp
from jax import lax
from jax.experimental import pallas as pl
from jax.experimental.pallas import tpu as pltpu
from jax.experimental.pallas import tpu_sc as plsc

NC = 2
NS = 16
L = 16
NW = NC * NS
N_TOK = 4 * 2048
D = 768
TOK_PER_W = N_TOK // NW
BUFT = 56
CHUNKS = (56, 56, 56, 56, 32)
DV = D // L

_mesh = plsc.VectorSubcoreMesh(core_axis_name="c", subcore_axis_name="s")


@functools.partial(
    pl.kernel,
    out_type=jax.ShapeDtypeStruct((N_TOK, D), jnp.float32),
    mesh=_mesh,
    scratch_types=[
        pltpu.VMEM((TOK_PER_W,), jnp.int32),
        pltpu.VMEM((4 * TOK_PER_W,), jnp.int32),
        pltpu.VMEM((BUFT, D), jnp.float32),
        pltpu.VMEM((BUFT, D), jnp.float32),
        pltpu.VMEM((BUFT, D), jnp.float32),
        pltpu.SemaphoreType.DMA,
        pltpu.SemaphoreType.DMA,
        pltpu.SemaphoreType.DMA,
        pltpu.SemaphoreType.DMA,
    ],
)
def _embed4(ids_hbm, cstack_hbm,
            word_hbm, p0_hbm, p1_hbm, p2_hbm, p3_hbm,
            out_hbm, idsb, cb4,
            acc, tmpa, tmpb,
            sem_w, sem_a, sem_b, sem_o):
    wid = lax.axis_index("s") * NC + lax.axis_index("c")
    wbase = wid * TOK_PER_W

    pltpu.sync_copy(ids_hbm.at[pl.ds(wbase, TOK_PER_W)], idsb)
    w_first = pltpu.async_copy(
        word_hbm.at[idsb.at[pl.ds(0, CHUNKS[0])]],
        acc.at[pl.ds(0, CHUNKS[0])], sem_w)
    pltpu.async_copy(cstack_hbm.at[pl.ds(4 * wbase, 4 * TOK_PER_W)],
                     cb4, sem_o).wait()

    def add_pass(accr, tmpr, rows):
        def row(t, c):
            for j in range(DV):
                sl = pl.ds(j * L, L)
                plsc.addupdate(accr.at[t, sl], tmpr[t, sl])
            return c
        lax.fori_loop(0, rows, row, 0)

    def final_pass(accr, tmpr, rows):
        def row(t, c):
            for j in range(DV):
                sl = pl.ds(j * L, L)
                tmpr[t, sl] = accr[t, sl] + tmpr[t, sl]
            return c
        lax.fori_loop(0, rows, row, 0)

    def fire_g0(off, n):
        return pltpu.async_copy(p0_hbm.at[cb4.at[pl.ds(off, n)]],
                                tmpa.at[pl.ds(0, n)], sem_a)

    def chunk_body(off, n, prev=None, w_pref=None, g0_pref=None, nxt=None):
        acc_s = acc.at[pl.ds(0, n)]
        ta_s = tmpa.at[pl.ds(0, n)]
        tb_s = tmpb.at[pl.ds(0, n)]
        out_dst = out_hbm.at[pl.ds(wbase + off, n)]
        if w_pref is None:
            w = pltpu.async_copy(word_hbm.at[idsb.at[pl.ds(off, n)]],
                                 acc_s, sem_w)
        else:
            w = w_pref
        g0 = g0_pref if g0_pref is not None else fire_g0(off, n)
        if prev is not None:
            po, pn = prev
            pltpu.make_async_copy(tmpb.at[pl.ds(0, pn)],
                                  out_hbm.at[pl.ds(wbase + po, pn)],
                                  sem_o).wait()
        g1 = pltpu.async_copy(p1_hbm.at[cb4.at[pl.ds(TOK_PER_W + off, n)]],
                              tb_s, sem_b)
        w.wait()
        g0.wait()
        add_pass(acc, tmpa, n)
        g2 = pltpu.async_copy(
            p2_hbm.at[cb4.at[pl.ds(2 * TOK_PER_W + off, n)]], ta_s, sem_a)
        g1.wait()
        add_pass(acc, tmpb, n)
        g3 = pltpu.async_copy(
            p3_hbm.at[cb4.at[pl.ds(3 * TOK_PER_W + off, n)]], tb_s, sem_b)
        g2.wait()
        add_pass(acc, tmpa, n)
        g0_next = fire_g0(*nxt) if nxt is not None else None
        g3.wait()
        final_pass(acc, tmpb, n)
        pltpu.async_copy(tb_s, out_dst, sem_o)
        return g0_next

    off = 0
    prev = None
    g0_next = None
    for k, n in enumerate(CHUNKS):
        nxt_off = off + n
        nxt = ((nxt_off, CHUNKS[k + 1])
               if k + 1 < len(CHUNKS) else None)
        g0_next = chunk_body(off, n, prev=prev,
                             w_pref=w_first if k == 0 else None,
                             g0_pref=g0_next, nxt=nxt)
        prev = (off, n)
        off += n

    po, pn = prev
    pltpu.make_async_copy(tmpb.at[pl.ds(0, pn)],
                          out_hbm.at[pl.ds(wbase + po, pn)], sem_o).wait()


def kernel(ids, coords, word, pos0, pos1, pos2, pos3):
    B, S = ids.shape
    ids_f = ids.reshape(N_TOK).astype(jnp.int32)
    cstack = (coords.reshape(NW, TOK_PER_W, 4).astype(jnp.int32)
              .transpose(0, 2, 1).reshape(4 * N_TOK))
    out = _embed4(ids_f, cstack, word, pos0, pos1, pos2, pos3)
    return out.reshape(B, S, D)
